# baseline (speedup 1.0000x reference)
import functools
import math

import jax
import jax.numpy as jnp
from jax import lax
from jax.experimental import pallas as pl
from jax.experimental.pallas import tpu as pltpu
from jax.experimental.pallas import tpu_sc as plsc

_BS = 256
_SUB = 8
_NC, _NS = 2, 16
_NW = _NC * _NS
_C = 64


def _tc_body(out_ref):
    i = pl.program_id(0)
    b, bs, e = out_ref.shape
    col = jax.lax.broadcasted_iota(jnp.int32, (_SUB, e), 1)
    parity = col & 1
    odd = parity == 1
    colf = (col - parity).astype(jnp.float32)
    freq = jnp.exp(colf * (-math.log(10000.0) / e))
    pos0 = (i * bs + jax.lax.broadcasted_iota(jnp.int32, (_SUB, e), 0)).astype(
        jnp.float32
    )
    ang = pos0 * freq
    s, c = jnp.sin(ang), jnp.cos(ang)
    p = jnp.where(odd, c, s)
    q = jnp.where(odd, -s, c)
    dang = freq * float(_SUB)
    sd, cd = jnp.sin(dang), jnp.cos(dang)
    for j in range(bs // _SUB):
        out_ref[:, j * _SUB : (j + 1) * _SUB, :] = jnp.broadcast_to(
            p[None], (b, _SUB, e)
        )
        p, q = p * cd + q * sd, q * cd - p * sd


def _sc_plane(pe, S, E):
    rw = S // _NW
    nchunks = rw // _C
    mesh = plsc.VectorSubcoreMesh(core_axis_name="c", subcore_axis_name="s")

    @functools.partial(
        pl.kernel,
        mesh=mesh,
        out_type=jax.ShapeDtypeStruct((S, E), jnp.float32),
        scratch_types=[
            pltpu.VMEM((2, _C, E), jnp.float32),
            pltpu.SemaphoreType.DMA,
            pltpu.SemaphoreType.DMA,
        ],
    )
    def k(pe_hbm, out_hbm, buf, in_sem, out_sem):
        wid = lax.axis_index("s") * _NC + lax.axis_index("c")
        base = wid * rw

        def fill(slot, j):
            return pltpu.make_async_copy(
                pe_hbm.at[pl.ds(base + j * _C, _C)], buf.at[slot], in_sem
            )

        def drain(slot, j):
            return pltpu.make_async_copy(
                buf.at[slot], out_hbm.at[pl.ds(base + j * _C, _C)], out_sem
            )

        fill(0, 0).start()
        for j in range(nchunks):
            slot = j & 1
            if j + 1 < nchunks:
                fill(1 - slot, j + 1).start()
            fill(slot, j).wait()
            drain(slot, j).start()
            drain(slot, j).wait()

    return k(pe)


def kernel(x, pe):
    B, S = x.shape
    _, E = pe.shape
    tc_part = pl.pallas_call(
        _tc_body,
        grid=(S // _BS,),
        out_specs=pl.BlockSpec((B - 1, _BS, E), lambda i: (0, i, 0)),
        out_shape=jax.ShapeDtypeStruct((B - 1, S, E), pe.dtype),
    )()
    sc_part = _sc_plane(pe, S, E)
    return jnp.concatenate([tc_part, sc_part[None]], axis=0)


# speedup vs baseline: 3.7726x; 3.7726x over previous
import math

import jax
import jax.numpy as jnp
from jax.experimental import pallas as pl

_BS = 256
_SUB = 8


def _body(out_ref):
    i = pl.program_id(0)
    b, bs, e = out_ref.shape
    col = jax.lax.broadcasted_iota(jnp.int32, (_SUB, e), 1)
    parity = col & 1
    odd = parity == 1
    colf = (col - parity).astype(jnp.float32)
    freq = jnp.exp(colf * (-math.log(10000.0) / e))
    pos0 = (i * bs + jax.lax.broadcasted_iota(jnp.int32, (_SUB, e), 0)).astype(
        jnp.float32
    )
    ang = pos0 * freq
    s, c = jnp.sin(ang), jnp.cos(ang)
    p = jnp.where(odd, c, s)
    q = jnp.where(odd, -s, c)
    dang = freq * float(_SUB)
    sd, cd = jnp.sin(dang), jnp.cos(dang)
    for j in range(bs // _SUB):
        out_ref[:, j * _SUB : (j + 1) * _SUB, :] = jnp.broadcast_to(
            p[None], (b, _SUB, e)
        )
        p, q = p * cd + q * sd, q * cd - p * sd


def kernel(x, pe):
    B, S = x.shape
    _, E = pe.shape
    return pl.pallas_call(
        _body,
        grid=(S // _BS,),
        out_specs=pl.BlockSpec((B, _BS, E), lambda i: (0, i, 0)),
        out_shape=jax.ShapeDtypeStruct((B, S, E), pe.dtype),
    )()
